# baseline (speedup 1.0000x reference)
import functools

import numpy as np
import jax
import jax.numpy as jnp
from jax import lax
from jax.experimental import pallas as pl
from jax.experimental.pallas import tpu as pltpu
from jax.experimental.pallas import tpu_sc as plsc

BATCH = 4096
KCLS = 8
MARGIN = 0.3
LANES = 16
NC = 2
NS = 16
NW = NC * NS
ROWS_PER_W = BATCH // NW
RSLAB = 8
NSLAB = ROWS_PER_W // RSLAB
CHUNKS = BATCH // LANES
UNROLL = 8
SUB = CHUNKS // UNROLL
BIG = np.float32(3.0e38)


def _sc_body(d_hbm, out_hbm, buf0, buf1, accv, sem0, sem1):
    c = lax.axis_index("c")
    s = lax.axis_index("s")
    wid = s * NC + c
    row0 = wid * ROWS_PER_W
    lane = lax.iota(jnp.int32, LANES)
    bufs = (buf0, buf1)
    sems = (sem0, sem1)

    def start(t, b):
        pltpu.async_copy(d_hbm.at[pl.ds(row0 + t * RSLAB, RSLAB)],
                         bufs[b], sems[b])

    def wait(t, b):
        pltpu.make_async_copy(d_hbm.at[pl.ds(row0 + t * RSLAB, RSLAB)],
                              bufs[b], sems[b]).wait()

    def process_slab(buf, r0, total):
        for r in range(RSLAB):
            i = r0 + r
            cs = i // LANES
            off = ((i // KCLS) % 2) * KCLS
            dlane = i % LANES
            v = buf[r, pl.ds(cs * LANES, LANES)]
            in_block = (lane >= off) & (lane < off + KCLS)
            mask_pos = in_block & (lane != dlane)
            pos = jnp.max(jnp.where(mask_pos, v, -BIG))
            buf[r, pl.ds(cs * LANES, LANES)] = jnp.where(mask_pos, BIG, v)

            def chunk_step(j, carry):
                new = []
                for k in range(UNROLL):
                    m1, m2 = carry[2 * k], carry[2 * k + 1]
                    w = buf[r, pl.ds((j + k * SUB) * LANES, LANES)]
                    new.append(jnp.minimum(m1, w))
                    new.append(jnp.minimum(m2, jnp.maximum(m1, w)))
                return tuple(new)

            init = tuple(jnp.full((LANES,), BIG, jnp.float32)
                         for _ in range(2 * UNROLL))
            acc = lax.fori_loop(0, SUB, chunk_step, init)
            m1, m2 = acc[0], acc[1]
            for k in range(1, UNROLL):
                y1, y2 = acc[2 * k], acc[2 * k + 1]
                m2 = jnp.minimum(jnp.maximum(m1, y1), jnp.minimum(m2, y2))
                m1 = jnp.minimum(m1, y1)

            g1 = jnp.min(m1)
            first = plsc.all_reduce_ffs(m1 == g1)
            u = jnp.where(lane == first, BIG, m1)
            g2 = jnp.minimum(jnp.min(u), jnp.min(m2))
            total = total + jnp.maximum(jnp.maximum(pos, 0.0) - g2 + MARGIN, 0.0)
        return total

    start(0, 0)

    def outer(h, total):
        t0 = 2 * h
        wait(t0, 0)
        start(t0 + 1, 1)
        total = process_slab(buf0, row0 + t0 * RSLAB, total)
        wait(t0 + 1, 1)

        @pl.when(t0 + 2 < NSLAB)
        def _():
            start(t0 + 2, 0)

        total = process_slab(buf1, row0 + (t0 + 1) * RSLAB, total)
        return total

    total = lax.fori_loop(0, NSLAB // 2, outer, jnp.float32(0.0))
    accv[...] = jnp.where(lane == 0, total, jnp.float32(0.0))
    pltpu.sync_copy(accv, out_hbm.at[wid])


@functools.partial(
    pl.kernel,
    out_type=jax.ShapeDtypeStruct((NW, LANES), jnp.float32),
    mesh=plsc.VectorSubcoreMesh(core_axis_name="c", subcore_axis_name="s"),
    scratch_types=[
        pltpu.VMEM((RSLAB, BATCH), jnp.float32),
        pltpu.VMEM((RSLAB, BATCH), jnp.float32),
        pltpu.VMEM((LANES,), jnp.float32),
        pltpu.SemaphoreType.DMA,
        pltpu.SemaphoreType.DMA,
    ],
    compiler_params=pltpu.CompilerParams(needs_layout_passes=False),
)
def _sc_triplet(d_hbm, out_hbm, buf0, buf1, accv, sem0, sem1):
    _sc_body(d_hbm, out_hbm, buf0, buf1, accv, sem0, sem1)


def kernel(distance_matrix):
    partials = _sc_triplet(distance_matrix)
    return jnp.sum(partials) / jnp.float32(BATCH)


# speedup vs baseline: 1.1898x; 1.1898x over previous
import functools

import numpy as np
import jax
import jax.numpy as jnp
from jax import lax
from jax.experimental import pallas as pl
from jax.experimental.pallas import tpu as pltpu
from jax.experimental.pallas import tpu_sc as plsc

BATCH = 4096
KCLS = 8
MARGIN = 0.3
LANES = 16
NC = 2
NS = 16
NW = NC * NS
TC_ROWS = 2048
SC_ROWS = BATCH - TC_ROWS
ROWS_PER_W = SC_ROWS // NW
RSLAB = 8
NSLAB = ROWS_PER_W // RSLAB
CHUNKS = BATCH // LANES
UNROLL = 4
SUB = CHUNKS // UNROLL
TC_BLK = 256
BIG = np.float32(3.0e38)


def _sc_body(d_hbm, out_hbm, buf0, buf1, accv, sem0, sem1):
    c = lax.axis_index("c")
    s = lax.axis_index("s")
    wid = s * NC + c
    row0 = TC_ROWS + wid * ROWS_PER_W
    lane = lax.iota(jnp.int32, LANES)
    bufs = (buf0, buf1)
    sems = (sem0, sem1)

    def start(t, b):
        pltpu.async_copy(d_hbm.at[pl.ds(row0 + t * RSLAB, RSLAB)],
                         bufs[b], sems[b])

    def wait(t, b):
        pltpu.make_async_copy(d_hbm.at[pl.ds(row0 + t * RSLAB, RSLAB)],
                              bufs[b], sems[b]).wait()

    def process_slab(buf, r0, total):
        for r in range(RSLAB):
            i = r0 + r
            cs = i // LANES
            off = ((i // KCLS) % 2) * KCLS
            dlane = i % LANES
            v = buf[r, pl.ds(cs * LANES, LANES)]
            in_block = (lane >= off) & (lane < off + KCLS)
            mask_pos = in_block & (lane != dlane)
            pos = jnp.max(jnp.where(mask_pos, v, -BIG))
            buf[r, pl.ds(cs * LANES, LANES)] = jnp.where(mask_pos, BIG, v)

            def chunk_step(j, carry):
                new = []
                for k in range(UNROLL):
                    m1, m2 = carry[2 * k], carry[2 * k + 1]
                    w = buf[r, pl.ds((j + k * SUB) * LANES, LANES)]
                    new.append(jnp.minimum(m1, w))
                    new.append(jnp.minimum(m2, jnp.maximum(m1, w)))
                return tuple(new)

            init = tuple(jnp.full((LANES,), BIG, jnp.float32)
                         for _ in range(2 * UNROLL))
            acc = lax.fori_loop(0, SUB, chunk_step, init)
            m1, m2 = acc[0], acc[1]
            for k in range(1, UNROLL):
                y1, y2 = acc[2 * k], acc[2 * k + 1]
                m2 = jnp.minimum(jnp.maximum(m1, y1), jnp.minimum(m2, y2))
                m1 = jnp.minimum(m1, y1)

            g1 = jnp.min(m1)
            first = plsc.all_reduce_ffs(m1 == g1)
            u = jnp.where(lane == first, BIG, m1)
            g2 = jnp.minimum(jnp.min(u), jnp.min(m2))
            total = total + jnp.maximum(jnp.maximum(pos, 0.0) - g2 + MARGIN, 0.0)
        return total

    start(0, 0)

    def outer(h, total):
        t0 = 2 * h
        wait(t0, 0)
        start(t0 + 1, 1)
        total = process_slab(buf0, row0 + t0 * RSLAB, total)
        wait(t0 + 1, 1)

        @pl.when(t0 + 2 < NSLAB)
        def _():
            start(t0 + 2, 0)

        total = process_slab(buf1, row0 + (t0 + 1) * RSLAB, total)
        return total

    total = lax.fori_loop(0, NSLAB // 2, outer, jnp.float32(0.0))
    accv[...] = jnp.where(lane == 0, total, jnp.float32(0.0))
    pltpu.sync_copy(accv, out_hbm.at[wid])


@functools.partial(
    pl.kernel,
    out_type=jax.ShapeDtypeStruct((NW, LANES), jnp.float32),
    mesh=plsc.VectorSubcoreMesh(core_axis_name="c", subcore_axis_name="s"),
    scratch_types=[
        pltpu.VMEM((RSLAB, BATCH), jnp.float32),
        pltpu.VMEM((RSLAB, BATCH), jnp.float32),
        pltpu.VMEM((LANES,), jnp.float32),
        pltpu.SemaphoreType.DMA,
        pltpu.SemaphoreType.DMA,
    ],
    compiler_params=pltpu.CompilerParams(needs_layout_passes=False),
)
def _sc_triplet(d_hbm, out_hbm, buf0, buf1, accv, sem0, sem1):
    _sc_body(d_hbm, out_hbm, buf0, buf1, accv, sem0, sem1)


def _tc_block(x_ref, o_ref):
    b = pl.program_id(0)
    x = x_ref[...]
    row = b * TC_BLK + lax.broadcasted_iota(jnp.int32, (TC_BLK, BATCH), 0)
    col = lax.broadcasted_iota(jnp.int32, (TC_BLK, BATCH), 1)
    z = col ^ row
    mask = (z < KCLS) & (z != 0)
    pos = jnp.max(jnp.where(mask, x, -BIG), axis=1, keepdims=True)
    xm = jnp.where(mask, BIG, x)
    m1 = jnp.min(xm, axis=1, keepdims=True)
    cmin = jnp.min(jnp.where(xm == m1, col, BATCH), axis=1, keepdims=True)
    m2 = jnp.min(jnp.where(col == cmin, BIG, xm), axis=1, keepdims=True)
    loss = jnp.maximum(jnp.maximum(pos, 0.0) - m2 + MARGIN, 0.0)
    o_ref[0, 0, 0] = jnp.sum(loss)


_tc_triplet = pl.pallas_call(
    _tc_block,
    grid=(TC_ROWS // TC_BLK,),
    in_specs=[pl.BlockSpec((TC_BLK, BATCH), lambda b: (b, 0))],
    out_specs=pl.BlockSpec((1, 1, 1), lambda b: (b, 0, 0), memory_space=pltpu.SMEM),
    out_shape=jax.ShapeDtypeStruct((TC_ROWS // TC_BLK, 1, 1), jnp.float32),
)


def kernel(distance_matrix):
    sc_parts = _sc_triplet(distance_matrix)
    tc_parts = _tc_triplet(distance_matrix)
    return (jnp.sum(sc_parts) + jnp.sum(tc_parts)) / jnp.float32(BATCH)
